# baseline (speedup 1.0000x reference)
import functools

import jax
import jax.numpy as jnp
from jax import lax
from jax.experimental import pallas as pl
from jax.experimental.pallas import tpu as pltpu
from jax.experimental.pallas import tpu_sc as plsc

N_NUM = 100
N_CAT = 26
VOCAB = 1000
EMBED_DIM = 128
BATCH = 16384

NUM_CORES = 2
NUM_SUBCORES = 16
NW = NUM_CORES * NUM_SUBCORES

PAD_CAT = 32
ROWS = BATCH * PAD_CAT
BLK = 128
IDX_ROWS = ROWS // BLK
IDX_PER_W = IDX_ROWS // NW
NBUF = 4


def _sc_gather(tables_flat, idx2d):
    mesh = plsc.VectorSubcoreMesh(core_axis_name="c", subcore_axis_name="s")

    @functools.partial(
        pl.kernel,
        mesh=mesh,
        out_type=jax.ShapeDtypeStruct((ROWS, EMBED_DIM), jnp.float32),
        scratch_types=[
            pltpu.VMEM((IDX_PER_W, BLK), jnp.int32),
            pltpu.VMEM((NBUF, BLK, EMBED_DIM), jnp.float32),
            [pltpu.SemaphoreType.DMA] * NBUF,
            [pltpu.SemaphoreType.DMA] * NBUF,
        ],
    )
    def k(tab_hbm, idx_hbm, out_hbm, idx_v, bufs, gsems, osems):
        wid = lax.axis_index("s") * NUM_CORES + lax.axis_index("c")
        base = wid * IDX_PER_W
        pltpu.sync_copy(idx_hbm.at[pl.ds(base, IDX_PER_W)], idx_v)
        lane = lax.iota(jnp.int32, 16)

        def flats(j):
            for c in range(BLK // 16):
                r0 = (base + j) * BLK + c * 16
                field = jnp.minimum((r0 + lane) & (PAD_CAT - 1), N_CAT - 1)
                idx_v[j, pl.ds(c * 16, 16)] = (
                    field * VOCAB + idx_v[j, pl.ds(c * 16, 16)]
                )

        def gather(j, b):
            return pltpu.make_async_copy(
                tab_hbm.at[idx_v.at[j]], bufs.at[b], gsems[b])

        def out_copy(j, b):
            return pltpu.make_async_copy(
                bufs.at[b], out_hbm.at[pl.ds((base + j) * BLK, BLK)],
                osems[b])

        for b in range(NBUF):
            flats(b)
            gather(b, b).start()

        def step(jo, carry):
            for b in range(NBUF):
                j = jo * NBUF + b
                gather(j, b).wait()
                out_copy(j, b).start()
                jn = j + NBUF

                @pl.when(jn < IDX_PER_W)
                def _():
                    out_copy(j, b).wait()
                    flats(jn)
                    gather(jn, b).start()

            return carry

        lax.fori_loop(0, IDX_PER_W // NBUF, step, 0)
        for b in range(NBUF):
            out_copy(IDX_PER_W - NBUF + b, b).wait()

    return k(tables_flat, idx2d)


def _mlp(x_num, W1, b1, W2, b2):
    BM = 1024

    def body(x_ref, w1_ref, b1_ref, w2_ref, b2_ref, o_ref):
        h = jnp.dot(x_ref[...], w1_ref[...],
                    preferred_element_type=jnp.float32) + b1_ref[...]
        h = jnp.maximum(h, 0.0)
        o_ref[...] = jnp.dot(h, w2_ref[...],
                             preferred_element_type=jnp.float32) + b2_ref[...]

    return pl.pallas_call(
        body,
        grid=(BATCH // BM,),
        in_specs=[
            pl.BlockSpec((BM, N_NUM), lambda i: (i, 0)),
            pl.BlockSpec((N_NUM, EMBED_DIM), lambda i: (0, 0)),
            pl.BlockSpec((1, EMBED_DIM), lambda i: (0, 0)),
            pl.BlockSpec((EMBED_DIM, EMBED_DIM), lambda i: (0, 0)),
            pl.BlockSpec((1, EMBED_DIM), lambda i: (0, 0)),
        ],
        out_specs=pl.BlockSpec((BM, EMBED_DIM), lambda i: (i, 0)),
        out_shape=jax.ShapeDtypeStruct((BATCH, EMBED_DIM), jnp.float32),
    )(x_num, W1, b1.reshape(1, EMBED_DIM), W2, b2.reshape(1, EMBED_DIM))


def kernel(x_num, x_cat, W1, b1, W2, b2, tables):
    idxp = jnp.pad(x_cat.astype(jnp.int32), ((0, 0), (0, PAD_CAT - N_CAT)))
    idx2d = idxp.reshape(IDX_ROWS, BLK)
    tables_flat = tables.reshape(N_CAT * VOCAB, EMBED_DIM)
    out = _sc_gather(tables_flat, idx2d)
    x_cats = out.reshape(BATCH, PAD_CAT, EMBED_DIM)[:, :N_CAT, :]
    num_out = _mlp(x_num, W1, b1, W2, b2)[:, None, :]
    return (num_out, x_cats)


# speedup vs baseline: 12.3014x; 12.3014x over previous
import functools

import jax
import jax.numpy as jnp
from jax import lax
from jax.experimental import pallas as pl
from jax.experimental.pallas import tpu as pltpu
from jax.experimental.pallas import tpu_sc as plsc

N_NUM = 100
N_CAT = 26
VOCAB = 1000
EMBED_DIM = 128
BATCH = 16384

NUM_CORES = 2
NUM_SUBCORES = 16
NW = NUM_CORES * NUM_SUBCORES

GB = 4
GROUP = GB * N_CAT
NGRP = BATCH // GB
GRP_PER_W = NGRP // NW
NBUF = 4
CHUNKS = (0, 16, 32, 48, 64, 80, 88)


def _sc_gather(tables_flat, idx2d):
    mesh = plsc.VectorSubcoreMesh(core_axis_name="c", subcore_axis_name="s")

    @functools.partial(
        pl.kernel,
        mesh=mesh,
        out_type=jax.ShapeDtypeStruct((BATCH, N_CAT, EMBED_DIM), jnp.float32),
        scratch_types=[
            pltpu.VMEM((GRP_PER_W, GROUP), jnp.int32),
            pltpu.VMEM((GRP_PER_W, GROUP), jnp.int32),
            pltpu.VMEM((GROUP,), jnp.int32),
            pltpu.VMEM((NBUF, GROUP, EMBED_DIM), jnp.float32),
            [pltpu.SemaphoreType.DMA] * NBUF,
            [pltpu.SemaphoreType.DMA] * NBUF,
        ],
    )
    def k(tab_hbm, idx_hbm, out_hbm, idx_v, flat_v, off_v, bufs,
          gsems, osems):
        wid = lax.axis_index("s") * NUM_CORES + lax.axis_index("c")
        base = wid * GRP_PER_W
        pltpu.sync_copy(idx_hbm.at[pl.ds(base, GRP_PER_W)], idx_v)
        lane = lax.iota(jnp.int32, 16)
        for o in CHUNKS:
            off_v[pl.ds(o, 16)] = lax.rem(o + lane, N_CAT) * VOCAB

        def flats(g):
            for o in CHUNKS:
                flat_v[g, pl.ds(o, 16)] = (
                    off_v[pl.ds(o, 16)] + idx_v[g, pl.ds(o, 16)]
                )

        def gather(g, b):
            return pltpu.make_async_copy(
                tab_hbm.at[flat_v.at[g]], bufs.at[b], gsems[b])

        def out_copies(g, b):
            b0 = (base + g) * GB
            return [
                pltpu.make_async_copy(
                    bufs.at[b, pl.ds(q * N_CAT, N_CAT)],
                    out_hbm.at[b0 + q], osems[b])
                for q in range(GB)
            ]

        for b in range(NBUF):
            flats(b)
            gather(b, b).start()

        def step(go, carry):
            for b in range(NBUF):
                g = go * NBUF + b
                gather(g, b).wait()
                for c in out_copies(g, b):
                    c.start()
                gn = g + NBUF

                @pl.when(gn < GRP_PER_W)
                def _():
                    for c in out_copies(g, b):
                        c.wait()
                    flats(gn)
                    gather(gn, b).start()

            return carry

        lax.fori_loop(0, GRP_PER_W // NBUF, step, 0)
        for b in range(NBUF):
            for c in out_copies(GRP_PER_W - NBUF + b, b):
                c.wait()

    return k(tables_flat, idx2d)


def _mlp(x_num, W1, b1, W2, b2):
    BM = 1024

    def body(x_ref, w1_ref, b1_ref, w2_ref, b2_ref, o_ref):
        h = jnp.dot(x_ref[...], w1_ref[...],
                    preferred_element_type=jnp.float32) + b1_ref[...]
        h = jnp.maximum(h, 0.0)
        o_ref[...] = jnp.dot(h, w2_ref[...],
                             preferred_element_type=jnp.float32) + b2_ref[...]

    return pl.pallas_call(
        body,
        grid=(BATCH // BM,),
        in_specs=[
            pl.BlockSpec((BM, N_NUM), lambda i: (i, 0)),
            pl.BlockSpec((N_NUM, EMBED_DIM), lambda i: (0, 0)),
            pl.BlockSpec((1, EMBED_DIM), lambda i: (0, 0)),
            pl.BlockSpec((EMBED_DIM, EMBED_DIM), lambda i: (0, 0)),
            pl.BlockSpec((1, EMBED_DIM), lambda i: (0, 0)),
        ],
        out_specs=pl.BlockSpec((BM, EMBED_DIM), lambda i: (i, 0)),
        out_shape=jax.ShapeDtypeStruct((BATCH, EMBED_DIM), jnp.float32),
    )(x_num, W1, b1.reshape(1, EMBED_DIM), W2, b2.reshape(1, EMBED_DIM))


def kernel(x_num, x_cat, W1, b1, W2, b2, tables):
    idx2d = x_cat.astype(jnp.int32).reshape(NGRP, GROUP)
    tables_flat = tables.reshape(N_CAT * VOCAB, EMBED_DIM)
    x_cats = _sc_gather(tables_flat, idx2d)
    num_out = _mlp(x_num, W1, b1, W2, b2)[:, None, :]
    return (num_out, x_cats)


# speedup vs baseline: 12.3326x; 1.0025x over previous
import functools

import jax
import jax.numpy as jnp
from jax import lax
from jax.experimental import pallas as pl
from jax.experimental.pallas import tpu as pltpu
from jax.experimental.pallas import tpu_sc as plsc

N_NUM = 100
N_CAT = 26
VOCAB = 1000
EMBED_DIM = 128
BATCH = 16384

NUM_CORES = 2
NUM_SUBCORES = 16
NW = NUM_CORES * NUM_SUBCORES

GB = 4
GROUP = GB * N_CAT
NGRP = BATCH // GB
GRP_PER_W = NGRP // NW
NBUF = 4
CHUNKS = (0, 16, 32, 48, 64, 80, 88)


def _sc_gather(tables_flat, idx2d):
    mesh = plsc.VectorSubcoreMesh(core_axis_name="c", subcore_axis_name="s")

    @functools.partial(
        pl.kernel,
        mesh=mesh,
        out_type=jax.ShapeDtypeStruct((BATCH, N_CAT, EMBED_DIM), jnp.float32),
        compiler_params=pltpu.CompilerParams(use_tc_tiling_on_sc=True),
        scratch_types=[
            pltpu.VMEM((GRP_PER_W, GROUP), jnp.int32),
            pltpu.VMEM((GRP_PER_W, GROUP), jnp.int32),
            pltpu.VMEM((GROUP,), jnp.int32),
            pltpu.VMEM((NBUF, GROUP, EMBED_DIM), jnp.float32),
            [pltpu.SemaphoreType.DMA] * NBUF,
            [pltpu.SemaphoreType.DMA] * NBUF,
        ],
    )
    def k(tab_hbm, idx_hbm, out_hbm, idx_v, flat_v, off_v, bufs,
          gsems, osems):
        wid = lax.axis_index("s") * NUM_CORES + lax.axis_index("c")
        base = wid * GRP_PER_W
        pltpu.sync_copy(idx_hbm.at[pl.ds(base, GRP_PER_W)], idx_v)
        lane = lax.iota(jnp.int32, 16)
        for o in CHUNKS:
            off_v[pl.ds(o, 16)] = lax.rem(o + lane, N_CAT) * VOCAB

        def flats(g):
            for o in CHUNKS:
                flat_v[g, pl.ds(o, 16)] = (
                    off_v[pl.ds(o, 16)] + idx_v[g, pl.ds(o, 16)]
                )

        def gather(g, b):
            return pltpu.make_async_copy(
                tab_hbm.at[flat_v.at[g]], bufs.at[b], gsems[b])

        def out_copies(g, b):
            b0 = (base + g) * GB
            return [
                pltpu.make_async_copy(
                    bufs.at[b, pl.ds(q * N_CAT, N_CAT)],
                    out_hbm.at[b0 + q], osems[b])
                for q in range(GB)
            ]

        for b in range(NBUF):
            flats(b)
            gather(b, b).start()

        def step(go, carry):
            for b in range(NBUF):
                g = go * NBUF + b
                gather(g, b).wait()
                for c in out_copies(g, b):
                    c.start()
                gn = g + NBUF

                @pl.when(gn < GRP_PER_W)
                def _():
                    for c in out_copies(g, b):
                        c.wait()
                    flats(gn)
                    gather(gn, b).start()

            return carry

        lax.fori_loop(0, GRP_PER_W // NBUF, step, 0)
        for b in range(NBUF):
            for c in out_copies(GRP_PER_W - NBUF + b, b):
                c.wait()

    return k(tables_flat, idx2d)


def _mlp(x_num, W1, b1, W2, b2):
    BM = 1024

    def body(x_ref, w1_ref, b1_ref, w2_ref, b2_ref, o_ref):
        h = jnp.dot(x_ref[...], w1_ref[...],
                    preferred_element_type=jnp.float32) + b1_ref[...]
        h = jnp.maximum(h, 0.0)
        o_ref[...] = jnp.dot(h, w2_ref[...],
                             preferred_element_type=jnp.float32) + b2_ref[...]

    return pl.pallas_call(
        body,
        grid=(BATCH // BM,),
        in_specs=[
            pl.BlockSpec((BM, N_NUM), lambda i: (i, 0)),
            pl.BlockSpec((N_NUM, EMBED_DIM), lambda i: (0, 0)),
            pl.BlockSpec((1, EMBED_DIM), lambda i: (0, 0)),
            pl.BlockSpec((EMBED_DIM, EMBED_DIM), lambda i: (0, 0)),
            pl.BlockSpec((1, EMBED_DIM), lambda i: (0, 0)),
        ],
        out_specs=pl.BlockSpec((BM, EMBED_DIM), lambda i: (i, 0)),
        out_shape=jax.ShapeDtypeStruct((BATCH, EMBED_DIM), jnp.float32),
    )(x_num, W1, b1.reshape(1, EMBED_DIM), W2, b2.reshape(1, EMBED_DIM))


def kernel(x_num, x_cat, W1, b1, W2, b2, tables):
    idx2d = x_cat.astype(jnp.int32).reshape(NGRP, GROUP)
    tables_flat = tables.reshape(N_CAT * VOCAB, EMBED_DIM)
    x_cats = _sc_gather(tables_flat, idx2d)
    num_out = _mlp(x_num, W1, b1, W2, b2)[:, None, :]
    return (num_out, x_cats)
